# Initial kernel scaffold; baseline (speedup 1.0000x reference)
import functools

import jax
import jax.numpy as jnp
from jax import lax
from jax.experimental import pallas as pl
from jax.experimental.pallas import tpu as pltpu
from jax.experimental.pallas import tpu_sc as plsc

N = 10000
E = 10000
P = 320000
DIN = 128
DH = 128
NCLS = 64

NC = 2
NS = 16
CH = 80
RT = N // NS

f32 = jnp.float32


def _sc_mesh():
    return plsc.VectorSubcoreMesh(
        core_axis_name="c", subcore_axis_name="s", num_cores=NC, num_subcores=NS
    )


def _make_deg():
    chunks = P // NC // NS // CH

    @functools.partial(
        pl.kernel,
        out_type=[
            jax.ShapeDtypeStruct((2 * N, 1), f32),
            jax.ShapeDtypeStruct((2 * N, 1), f32),
        ],
        mesh=_sc_mesh(),
        scratch_types=[
            pltpu.VMEM((CH,), jnp.int32),
            pltpu.VMEM((CH,), jnp.int32),
            pltpu.VMEM((CH, 1), f32),
            pltpu.VMEM_SHARED((N, 1), f32),
            pltpu.VMEM_SHARED((N, 1), f32),
        ],
    )
    def deg(vidx_hbm, eidx_hbm, zeros_hbm, ones_hbm, dv_out, de_out,
            vbuf, ebuf, ones_v, dv_s, de_s):
        c = lax.axis_index("c")
        s = lax.axis_index("s")
        r0 = s * RT
        pltpu.sync_copy(ones_hbm, ones_v)
        pltpu.sync_copy(zeros_hbm.at[pl.ds(r0, RT)], dv_s.at[pl.ds(r0, RT)])
        pltpu.sync_copy(zeros_hbm.at[pl.ds(r0, RT)], de_s.at[pl.ds(r0, RT)])
        plsc.subcore_barrier()

        base0 = c * (P // NC) + s * (P // NC // NS)

        @pl.loop(0, chunks)
        def _(i):
            b = base0 + i * CH
            pltpu.sync_copy(vidx_hbm.at[pl.ds(b, CH)], vbuf)
            pltpu.sync_copy(eidx_hbm.at[pl.ds(b, CH)], ebuf)
            pltpu.sync_copy(ones_v, dv_s.at[vbuf], add=True)
            pltpu.sync_copy(ones_v, de_s.at[ebuf], add=True)

        plsc.subcore_barrier()
        pltpu.sync_copy(dv_s.at[pl.ds(r0, RT)], dv_out.at[pl.ds(c * N + r0, RT)])
        pltpu.sync_copy(de_s.at[pl.ds(r0, RT)], de_out.at[pl.ds(c * N + r0, RT)])

    return deg


_deg = _make_deg()


def _make_gs(D):
    chunks = P // NS // CH

    @functools.partial(
        pl.kernel,
        out_type=jax.ShapeDtypeStruct((2 * N, D), f32),
        mesh=_sc_mesh(),
        scratch_types=[
            pltpu.VMEM((CH,), jnp.int32),
            pltpu.VMEM((CH,), jnp.int32),
            pltpu.VMEM((CH, D), f32),
            pltpu.SemaphoreType.DMA,
            pltpu.VMEM_SHARED((N, D), f32),
        ],
    )
    def gs(table_hbm, gidx_hbm, sidx_hbm, zeros_hbm, o_hbm,
           gbuf, sbuf, rows, sem, acc):
        c = lax.axis_index("c")
        s = lax.axis_index("s")
        r0 = s * RT
        pltpu.sync_copy(zeros_hbm.at[pl.ds(r0, RT)], acc.at[pl.ds(r0, RT)])
        plsc.subcore_barrier()

        base0 = s * (P // NS)

        @pl.loop(0, chunks)
        def _(i):
            b = base0 + i * CH
            pltpu.sync_copy(gidx_hbm.at[pl.ds(b, CH)], gbuf)
            pltpu.sync_copy(sidx_hbm.at[pl.ds(b, CH)], sbuf)
            for j in range(CH // 16):
                gbuf[pl.ds(16 * j, 16)] = gbuf[pl.ds(16 * j, 16)] + c * N
            pltpu.async_copy(table_hbm.at[gbuf], rows, sem).wait()
            pltpu.sync_copy(rows, acc.at[sbuf], add=True)

        plsc.subcore_barrier()
        pltpu.sync_copy(acc.at[pl.ds(r0, RT)], o_hbm.at[pl.ds(c * N + r0, RT)])

    return gs


_gs64 = _make_gs(64)
_gs32 = _make_gs(32)


def _combine_body(dvp, dep, dvis, dei):
    dv = dvp[0] + dvp[1]
    de = dep[0] + dep[1]
    dvis[...] = jnp.where(dv > 0, lax.rsqrt(dv), 0.0)
    dei[...] = jnp.where(de > 0, 1.0 / de, 0.0)


_combine = pl.pallas_call(
    _combine_body,
    grid=(8,),
    in_specs=[
        pl.BlockSpec((2, N // 8, 1), lambda i: (0, i, 0)),
        pl.BlockSpec((2, N // 8, 1), lambda i: (0, i, 0)),
    ],
    out_specs=[
        pl.BlockSpec((N // 8, 1), lambda i: (i, 0)),
        pl.BlockSpec((N // 8, 1), lambda i: (i, 0)),
    ],
    out_shape=[
        jax.ShapeDtypeStruct((N, 1), f32),
        jax.ShapeDtypeStruct((N, 1), f32),
    ],
)

_BR = 1000


def _lin0_body(x, w, b, dvis, o):
    h = jnp.dot(x[...], w[...], preferred_element_type=f32) + b[...]
    hs = h * dvis[...]
    o[0] = hs[:, :64]
    o[1] = hs[:, 64:]


_lin0 = pl.pallas_call(
    _lin0_body,
    grid=(N // _BR,),
    in_specs=[
        pl.BlockSpec((_BR, DIN), lambda i: (i, 0)),
        pl.BlockSpec((DIN, DH), lambda i: (0, 0)),
        pl.BlockSpec((1, DH), lambda i: (0, 0)),
        pl.BlockSpec((_BR, 1), lambda i: (i, 0)),
    ],
    out_specs=pl.BlockSpec((2, _BR, 64), lambda i: (0, i, 0)),
    out_shape=jax.ShapeDtypeStruct((2, N, 64), f32),
)


def _make_escale(D):
    def body(ye, dei, o):
        o[...] = ye[...] * dei[...][None]

    return pl.pallas_call(
        body,
        grid=(N // _BR,),
        in_specs=[
            pl.BlockSpec((2, _BR, D), lambda i: (0, i, 0)),
            pl.BlockSpec((_BR, 1), lambda i: (i, 0)),
        ],
        out_specs=pl.BlockSpec((2, _BR, D), lambda i: (0, i, 0)),
        out_shape=jax.ShapeDtypeStruct((2, N, D), f32),
    )


_escale64 = _make_escale(64)
_escale32 = _make_escale(32)


def _lin1_body(ho, w, b, dvis, o):
    sm = jnp.concatenate([ho[0], ho[1]], axis=1) * dvis[...]
    emb = jnp.maximum(sm, 0.0)
    h2 = jnp.dot(emb, w[...], preferred_element_type=f32) + b[...]
    hs2 = h2 * dvis[...]
    o[0] = hs2[:, :32]
    o[1] = hs2[:, 32:]


_lin1 = pl.pallas_call(
    _lin1_body,
    grid=(N // _BR,),
    in_specs=[
        pl.BlockSpec((2, _BR, 64), lambda i: (0, i, 0)),
        pl.BlockSpec((DH, NCLS), lambda i: (0, 0)),
        pl.BlockSpec((1, NCLS), lambda i: (0, 0)),
        pl.BlockSpec((_BR, 1), lambda i: (i, 0)),
    ],
    out_specs=pl.BlockSpec((2, _BR, 32), lambda i: (0, i, 0)),
    out_shape=jax.ShapeDtypeStruct((2, N, 32), f32),
)


def _final_body(ho, dvis, o):
    o[...] = jnp.concatenate([ho[0], ho[1]], axis=1) * dvis[...]


_final = pl.pallas_call(
    _final_body,
    grid=(N // _BR,),
    in_specs=[
        pl.BlockSpec((2, _BR, 32), lambda i: (0, i, 0)),
        pl.BlockSpec((_BR, 1), lambda i: (i, 0)),
    ],
    out_specs=pl.BlockSpec((_BR, NCLS), lambda i: (i, 0)),
    out_shape=jax.ShapeDtypeStruct((N, NCLS), f32),
)


def kernel(X, incidence, W0, b0, W1, b1):
    v_idx = incidence[0]
    e_idx = incidence[1]
    zN1 = jnp.zeros((N, 1), f32)
    ones_ch = jnp.ones((CH, 1), f32)
    z64 = jnp.zeros((N, 64), f32)
    z32 = jnp.zeros((N, 32), f32)

    dvp, dep = _deg(v_idx, e_idx, zN1, ones_ch)
    dvis, dei = _combine(dvp.reshape(2, N, 1), dep.reshape(2, N, 1))

    hs = _lin0(X, W0, b0.reshape(1, DH), dvis)
    ye = _gs64(hs.reshape(2 * N, 64), v_idx, e_idx, z64)
    yes = _escale64(ye.reshape(2, N, 64), dei)
    ho = _gs64(yes.reshape(2 * N, 64), e_idx, v_idx, z64)

    hs2 = _lin1(ho.reshape(2, N, 64), W1, b1.reshape(1, NCLS), dvis)
    ye2 = _gs32(hs2.reshape(2 * N, 32), v_idx, e_idx, z32)
    yes2 = _escale32(ye2.reshape(2, N, 32), dei)
    ho2 = _gs32(yes2.reshape(2 * N, 32), e_idx, v_idx, z32)

    return _final(ho2.reshape(2, N, 32), dvis)


# speedup vs baseline: 2.8491x; 2.8491x over previous
import functools

import jax
import jax.numpy as jnp
from jax import lax
from jax.experimental import pallas as pl
from jax.experimental.pallas import tpu as pltpu
from jax.experimental.pallas import tpu_sc as plsc

N = 10000
E = 10000
P = 320000
DIN = 128
DH = 128
NCLS = 64

NC = 2
NS = 16
CH = 80
DEGW = 16

f32 = jnp.float32


def _sc_mesh():
    return plsc.VectorSubcoreMesh(
        core_axis_name="c", subcore_axis_name="s", num_cores=NC, num_subcores=NS
    )


def _each_tile_rows(s, fn):
    @pl.when(s < NS - 1)
    def _():
        fn(pl.multiple_of(s * 624, 8), 624)

    @pl.when(s == NS - 1)
    def _():
        fn(624 * (NS - 1), N - 624 * (NS - 1))


def _make_deg():
    chunks = P // NC // NS // CH

    @functools.partial(
        pl.kernel,
        out_type=[
            jax.ShapeDtypeStruct((2 * N, DEGW), f32),
            jax.ShapeDtypeStruct((2 * N, DEGW), f32),
        ],
        mesh=_sc_mesh(),
        scratch_types=[
            pltpu.VMEM((CH,), jnp.int32),
            pltpu.VMEM((CH,), jnp.int32),
            pltpu.VMEM((CH, DEGW), f32),
            pltpu.VMEM_SHARED((N, DEGW), f32),
            pltpu.VMEM_SHARED((N, DEGW), f32),
        ],
        compiler_params=pltpu.CompilerParams(use_tc_tiling_on_sc=False),
    )
    def deg(vidx_hbm, eidx_hbm, zeros_hbm, ones_hbm, dv_out, de_out,
            vbuf, ebuf, ones_v, dv_s, de_s):
        c = lax.axis_index("c")
        s = lax.axis_index("s")
        pltpu.sync_copy(ones_hbm, ones_v)

        def zero_fn(off, sz):
            pltpu.sync_copy(zeros_hbm.at[pl.ds(off, sz)], dv_s.at[pl.ds(off, sz)])
            pltpu.sync_copy(zeros_hbm.at[pl.ds(off, sz)], de_s.at[pl.ds(off, sz)])

        _each_tile_rows(s, zero_fn)
        plsc.subcore_barrier()

        base0 = c * (P // NC) + s * (P // NC // NS)

        @pl.loop(0, chunks)
        def _(i):
            b = base0 + i * CH
            pltpu.sync_copy(vidx_hbm.at[pl.ds(b, CH)], vbuf)
            pltpu.sync_copy(eidx_hbm.at[pl.ds(b, CH)], ebuf)
            pltpu.sync_copy(ones_v, dv_s.at[vbuf], add=True)
            pltpu.sync_copy(ones_v, de_s.at[ebuf], add=True)

        plsc.subcore_barrier()

        def out_fn(off, sz):
            dst = pl.multiple_of(c * N + off, 8)
            pltpu.sync_copy(dv_s.at[pl.ds(off, sz)], dv_out.at[pl.ds(dst, sz)])
            pltpu.sync_copy(de_s.at[pl.ds(off, sz)], de_out.at[pl.ds(dst, sz)])

        _each_tile_rows(s, out_fn)

    return deg


_deg = _make_deg()


def _make_gs(D):
    chunks = P // NS // CH

    @functools.partial(
        pl.kernel,
        out_type=jax.ShapeDtypeStruct((2 * N, D), f32),
        mesh=_sc_mesh(),
        scratch_types=[
            pltpu.VMEM((CH,), jnp.int32),
            pltpu.VMEM((CH,), jnp.int32),
            pltpu.VMEM((CH, D), f32),
            pltpu.SemaphoreType.DMA,
            pltpu.VMEM_SHARED((N, D), f32),
        ],
        compiler_params=pltpu.CompilerParams(use_tc_tiling_on_sc=False),
    )
    def gs(table_hbm, gidx_hbm, sidx_hbm, zeros_hbm, o_hbm,
           gbuf, sbuf, rows, sem, acc):
        c = lax.axis_index("c")
        s = lax.axis_index("s")

        def zero_fn(off, sz):
            pltpu.sync_copy(zeros_hbm.at[pl.ds(off, sz)], acc.at[pl.ds(off, sz)])

        _each_tile_rows(s, zero_fn)
        plsc.subcore_barrier()

        base0 = s * (P // NS)

        @pl.loop(0, chunks)
        def _(i):
            b = base0 + i * CH
            pltpu.sync_copy(gidx_hbm.at[pl.ds(b, CH)], gbuf)
            pltpu.sync_copy(sidx_hbm.at[pl.ds(b, CH)], sbuf)
            for j in range(CH // 16):
                gbuf[pl.ds(16 * j, 16)] = gbuf[pl.ds(16 * j, 16)] + c * N
            pltpu.async_copy(table_hbm.at[gbuf], rows, sem).wait()
            pltpu.sync_copy(rows, acc.at[sbuf], add=True)

        plsc.subcore_barrier()

        def out_fn(off, sz):
            dst = pl.multiple_of(c * N + off, 8)
            pltpu.sync_copy(acc.at[pl.ds(off, sz)], o_hbm.at[pl.ds(dst, sz)])

        _each_tile_rows(s, out_fn)

    return gs


_gs64 = _make_gs(64)
_gs32 = _make_gs(32)


def _combine_body(dvp, dep, dvis, dei):
    dv = dvp[0, :, 0:1] + dvp[1, :, 0:1]
    de = dep[0, :, 0:1] + dep[1, :, 0:1]
    dvis[...] = jnp.where(dv > 0, lax.rsqrt(dv), 0.0)
    dei[...] = jnp.where(de > 0, 1.0 / de, 0.0)


_combine = pl.pallas_call(
    _combine_body,
    grid=(10,),
    in_specs=[
        pl.BlockSpec((2, N // 10, DEGW), lambda i: (0, i, 0)),
        pl.BlockSpec((2, N // 10, DEGW), lambda i: (0, i, 0)),
    ],
    out_specs=[
        pl.BlockSpec((N // 10, 1), lambda i: (i, 0)),
        pl.BlockSpec((N // 10, 1), lambda i: (i, 0)),
    ],
    out_shape=[
        jax.ShapeDtypeStruct((N, 1), f32),
        jax.ShapeDtypeStruct((N, 1), f32),
    ],
)

_BR = 1000


def _lin0_body(x, w, b, dvis, o):
    h = jnp.dot(x[...], w[...], preferred_element_type=f32) + b[...]
    hs = h * dvis[...]
    o[0] = hs[:, :64]
    o[1] = hs[:, 64:]


_lin0 = pl.pallas_call(
    _lin0_body,
    grid=(N // _BR,),
    in_specs=[
        pl.BlockSpec((_BR, DIN), lambda i: (i, 0)),
        pl.BlockSpec((DIN, DH), lambda i: (0, 0)),
        pl.BlockSpec((1, DH), lambda i: (0, 0)),
        pl.BlockSpec((_BR, 1), lambda i: (i, 0)),
    ],
    out_specs=pl.BlockSpec((2, _BR, 64), lambda i: (0, i, 0)),
    out_shape=jax.ShapeDtypeStruct((2, N, 64), f32),
)


def _make_escale(D):
    def body(ye, dei, o):
        o[...] = ye[...] * dei[...][None]

    return pl.pallas_call(
        body,
        grid=(N // _BR,),
        in_specs=[
            pl.BlockSpec((2, _BR, D), lambda i: (0, i, 0)),
            pl.BlockSpec((_BR, 1), lambda i: (i, 0)),
        ],
        out_specs=pl.BlockSpec((2, _BR, D), lambda i: (0, i, 0)),
        out_shape=jax.ShapeDtypeStruct((2, N, D), f32),
    )


_escale64 = _make_escale(64)
_escale32 = _make_escale(32)


def _lin1_body(ho, w, b, dvis, o):
    sm = jnp.concatenate([ho[0], ho[1]], axis=1) * dvis[...]
    emb = jnp.maximum(sm, 0.0)
    h2 = jnp.dot(emb, w[...], preferred_element_type=f32) + b[...]
    hs2 = h2 * dvis[...]
    o[0] = hs2[:, :32]
    o[1] = hs2[:, 32:]


_lin1 = pl.pallas_call(
    _lin1_body,
    grid=(N // _BR,),
    in_specs=[
        pl.BlockSpec((2, _BR, 64), lambda i: (0, i, 0)),
        pl.BlockSpec((DH, NCLS), lambda i: (0, 0)),
        pl.BlockSpec((1, NCLS), lambda i: (0, 0)),
        pl.BlockSpec((_BR, 1), lambda i: (i, 0)),
    ],
    out_specs=pl.BlockSpec((2, _BR, 32), lambda i: (0, i, 0)),
    out_shape=jax.ShapeDtypeStruct((2, N, 32), f32),
)


def _final_body(ho, dvis, o):
    o[...] = jnp.concatenate([ho[0], ho[1]], axis=1) * dvis[...]


_final = pl.pallas_call(
    _final_body,
    grid=(N // _BR,),
    in_specs=[
        pl.BlockSpec((2, _BR, 32), lambda i: (0, i, 0)),
        pl.BlockSpec((_BR, 1), lambda i: (i, 0)),
    ],
    out_specs=pl.BlockSpec((_BR, NCLS), lambda i: (i, 0)),
    out_shape=jax.ShapeDtypeStruct((N, NCLS), f32),
)


def kernel(X, incidence, W0, b0, W1, b1):
    v_idx = incidence[0]
    e_idx = incidence[1]
    zNd = jnp.zeros((N, DEGW), f32)
    ones_ch = jnp.ones((CH, DEGW), f32)
    z64 = jnp.zeros((N, 64), f32)
    z32 = jnp.zeros((N, 32), f32)

    dvp, dep = _deg(v_idx, e_idx, zNd, ones_ch)
    dvis, dei = _combine(dvp.reshape(2, N, DEGW), dep.reshape(2, N, DEGW))

    hs = _lin0(X, W0, b0.reshape(1, DH), dvis)
    ye = _gs64(hs.reshape(2 * N, 64), v_idx, e_idx, z64)
    yes = _escale64(ye.reshape(2, N, 64), dei)
    ho = _gs64(yes.reshape(2 * N, 64), e_idx, v_idx, z64)

    hs2 = _lin1(ho.reshape(2, N, 64), W1, b1.reshape(1, NCLS), dvis)
    ye2 = _gs32(hs2.reshape(2 * N, 32), v_idx, e_idx, z32)
    yes2 = _escale32(ye2.reshape(2, N, 32), dei)
    ho2 = _gs32(yes2.reshape(2 * N, 32), e_idx, v_idx, z32)

    return _final(ho2.reshape(2, N, 32), dvis)


# speedup vs baseline: 8.2271x; 2.8876x over previous
import functools

import jax
import jax.numpy as jnp
from jax import lax
from jax.experimental import pallas as pl
from jax.experimental.pallas import tpu as pltpu
from jax.experimental.pallas import tpu_sc as plsc

N = 10000
E = 10000
P = 320000
DIN = 128
DH = 128
NCLS = 64

NC = 2
NS = 16
CH = 80
DEGW = 16

f32 = jnp.float32


def _sc_mesh():
    return plsc.VectorSubcoreMesh(
        core_axis_name="c", subcore_axis_name="s", num_cores=NC, num_subcores=NS
    )


def _each_tile_rows(s, fn):
    @pl.when(s < NS - 1)
    def _():
        fn(pl.multiple_of(s * 624, 8), 624)

    @pl.when(s == NS - 1)
    def _():
        fn(624 * (NS - 1), N - 624 * (NS - 1))


def _make_deg():
    chunks = P // NC // NS // CH

    @functools.partial(
        pl.kernel,
        out_type=[
            jax.ShapeDtypeStruct((2 * N, DEGW), f32),
            jax.ShapeDtypeStruct((2 * N, DEGW), f32),
        ],
        mesh=_sc_mesh(),
        scratch_types=[
            pltpu.VMEM((CH,), jnp.int32),
            pltpu.VMEM((CH,), jnp.int32),
            pltpu.VMEM((CH, DEGW), f32),
            pltpu.VMEM_SHARED((N, DEGW), f32),
            pltpu.VMEM_SHARED((N, DEGW), f32),
        ],
        compiler_params=pltpu.CompilerParams(use_tc_tiling_on_sc=False),
    )
    def deg(vidx_hbm, eidx_hbm, zeros_hbm, ones_hbm, dv_out, de_out,
            vbuf, ebuf, ones_v, dv_s, de_s):
        c = lax.axis_index("c")
        s = lax.axis_index("s")
        pltpu.sync_copy(ones_hbm, ones_v)

        def zero_fn(off, sz):
            pltpu.sync_copy(zeros_hbm.at[pl.ds(off, sz)], dv_s.at[pl.ds(off, sz)])
            pltpu.sync_copy(zeros_hbm.at[pl.ds(off, sz)], de_s.at[pl.ds(off, sz)])

        _each_tile_rows(s, zero_fn)
        plsc.subcore_barrier()

        base0 = c * (P // NC) + s * (P // NC // NS)

        @pl.loop(0, chunks)
        def _(i):
            b = base0 + i * CH
            pltpu.sync_copy(vidx_hbm.at[pl.ds(b, CH)], vbuf)
            pltpu.sync_copy(eidx_hbm.at[pl.ds(b, CH)], ebuf)
            pltpu.sync_copy(ones_v, dv_s.at[vbuf], add=True)
            pltpu.sync_copy(ones_v, de_s.at[ebuf], add=True)

        plsc.subcore_barrier()

        def out_fn(off, sz):
            dst = pl.multiple_of(c * N + off, 8)
            pltpu.sync_copy(dv_s.at[pl.ds(off, sz)], dv_out.at[pl.ds(dst, sz)])
            pltpu.sync_copy(de_s.at[pl.ds(off, sz)], de_out.at[pl.ds(dst, sz)])

        _each_tile_rows(s, out_fn)

    return deg


_deg = _make_deg()


CPT = P // NS // CH


def _make_gs(D, NG):
    @functools.partial(
        pl.kernel,
        out_type=jax.ShapeDtypeStruct((2 * N, D), f32),
        mesh=_sc_mesh(),
        scratch_types=[
            pltpu.VMEM((2, NG, CH), jnp.int32),
            pltpu.VMEM((2, NG, CH), jnp.int32),
            pltpu.VMEM((2, NG, CH, D), f32),
            pltpu.SemaphoreType.DMA,
            pltpu.SemaphoreType.DMA,
            pltpu.VMEM_SHARED((N, D), f32),
        ],
        compiler_params=pltpu.CompilerParams(use_tc_tiling_on_sc=False),
    )
    def gs(table_hbm, gidx_hbm, sidx_hbm, zeros_hbm, o_hbm,
           gi, si, rows, sem0, sem1, acc):
        GRP = CPT // NG
        c = lax.axis_index("c")
        s = lax.axis_index("s")
        sems = (sem0, sem1)

        def zero_fn(off, sz):
            pltpu.sync_copy(zeros_hbm.at[pl.ds(off, sz)], acc.at[pl.ds(off, sz)])

        _each_tile_rows(s, zero_fn)
        plsc.subcore_barrier()

        row0 = s * CPT

        def fire(g, p):
            base = row0 + g * NG
            pltpu.sync_copy(gidx_hbm.at[pl.ds(base, NG)], gi.at[p])
            pltpu.sync_copy(sidx_hbm.at[pl.ds(base, NG)], si.at[p])

            @pl.when(c == 1)
            def _():
                for r in range(NG):
                    for j in range(CH // 16):
                        gi[p, r, pl.ds(16 * j, 16)] = (
                            gi[p, r, pl.ds(16 * j, 16)] + N
                        )

            for k in range(NG):
                pltpu.async_copy(table_hbm.at[gi.at[p, k]], rows.at[p, k], sems[p])

        def consume(p):
            for k in range(NG):
                pltpu.make_async_copy(
                    table_hbm.at[gi.at[p, k]], rows.at[p, k], sems[p]
                ).wait()
            for k in range(NG):
                pltpu.sync_copy(rows.at[p, k], acc.at[si.at[p, k]], add=True)

        fire(0, 0)

        @pl.loop(0, (GRP + 1) // 2)
        def _(i):
            g = 2 * i

            @pl.when(g + 1 < GRP)
            def _():
                fire(g + 1, 1)

            consume(0)

            @pl.when(g + 2 < GRP)
            def _():
                fire(g + 2, 0)

            @pl.when(g + 1 < GRP)
            def _():
                consume(1)

        plsc.subcore_barrier()

        def out_fn(off, sz):
            dst = pl.multiple_of(c * N + off, 8)
            pltpu.sync_copy(acc.at[pl.ds(off, sz)], o_hbm.at[pl.ds(dst, sz)])

        _each_tile_rows(s, out_fn)

    return gs


_gs64 = _make_gs(64, 5)
_gs32 = _make_gs(32, 10)


def _combine_body(dvp, dep, dvis, dei):
    dv = dvp[0, :, 0:1] + dvp[1, :, 0:1]
    de = dep[0, :, 0:1] + dep[1, :, 0:1]
    dvis[...] = jnp.where(dv > 0, lax.rsqrt(dv), 0.0)
    dei[...] = jnp.where(de > 0, 1.0 / de, 0.0)


_combine = pl.pallas_call(
    _combine_body,
    grid=(10,),
    in_specs=[
        pl.BlockSpec((2, N // 10, DEGW), lambda i: (0, i, 0)),
        pl.BlockSpec((2, N // 10, DEGW), lambda i: (0, i, 0)),
    ],
    out_specs=[
        pl.BlockSpec((N // 10, 1), lambda i: (i, 0)),
        pl.BlockSpec((N // 10, 1), lambda i: (i, 0)),
    ],
    out_shape=[
        jax.ShapeDtypeStruct((N, 1), f32),
        jax.ShapeDtypeStruct((N, 1), f32),
    ],
)

_BR = 1000


def _lin0_body(x, w, b, dvis, o):
    h = jnp.dot(x[...], w[...], preferred_element_type=f32) + b[...]
    hs = h * dvis[...]
    o[0] = hs[:, :64]
    o[1] = hs[:, 64:]


_lin0 = pl.pallas_call(
    _lin0_body,
    grid=(N // _BR,),
    in_specs=[
        pl.BlockSpec((_BR, DIN), lambda i: (i, 0)),
        pl.BlockSpec((DIN, DH), lambda i: (0, 0)),
        pl.BlockSpec((1, DH), lambda i: (0, 0)),
        pl.BlockSpec((_BR, 1), lambda i: (i, 0)),
    ],
    out_specs=pl.BlockSpec((2, _BR, 64), lambda i: (0, i, 0)),
    out_shape=jax.ShapeDtypeStruct((2, N, 64), f32),
)


def _make_escale(D):
    def body(ye, dei, o):
        o[...] = ye[...] * dei[...][None]

    return pl.pallas_call(
        body,
        grid=(N // _BR,),
        in_specs=[
            pl.BlockSpec((2, _BR, D), lambda i: (0, i, 0)),
            pl.BlockSpec((_BR, 1), lambda i: (i, 0)),
        ],
        out_specs=pl.BlockSpec((2, _BR, D), lambda i: (0, i, 0)),
        out_shape=jax.ShapeDtypeStruct((2, N, D), f32),
    )


_escale64 = _make_escale(64)
_escale32 = _make_escale(32)


def _lin1_body(ho, w, b, dvis, o):
    sm = jnp.concatenate([ho[0], ho[1]], axis=1) * dvis[...]
    emb = jnp.maximum(sm, 0.0)
    h2 = jnp.dot(emb, w[...], preferred_element_type=f32) + b[...]
    hs2 = h2 * dvis[...]
    o[0] = hs2[:, :32]
    o[1] = hs2[:, 32:]


_lin1 = pl.pallas_call(
    _lin1_body,
    grid=(N // _BR,),
    in_specs=[
        pl.BlockSpec((2, _BR, 64), lambda i: (0, i, 0)),
        pl.BlockSpec((DH, NCLS), lambda i: (0, 0)),
        pl.BlockSpec((1, NCLS), lambda i: (0, 0)),
        pl.BlockSpec((_BR, 1), lambda i: (i, 0)),
    ],
    out_specs=pl.BlockSpec((2, _BR, 32), lambda i: (0, i, 0)),
    out_shape=jax.ShapeDtypeStruct((2, N, 32), f32),
)


def _final_body(ho, dvis, o):
    o[...] = jnp.concatenate([ho[0], ho[1]], axis=1) * dvis[...]


_final = pl.pallas_call(
    _final_body,
    grid=(N // _BR,),
    in_specs=[
        pl.BlockSpec((2, _BR, 32), lambda i: (0, i, 0)),
        pl.BlockSpec((_BR, 1), lambda i: (i, 0)),
    ],
    out_specs=pl.BlockSpec((_BR, NCLS), lambda i: (i, 0)),
    out_shape=jax.ShapeDtypeStruct((N, NCLS), f32),
)


def kernel(X, incidence, W0, b0, W1, b1):
    v_idx = incidence[0]
    e_idx = incidence[1]
    zNd = jnp.zeros((N, DEGW), f32)
    ones_ch = jnp.ones((CH, DEGW), f32)
    z64 = jnp.zeros((N, 64), f32)
    z32 = jnp.zeros((N, 32), f32)

    dvp, dep = _deg(v_idx, e_idx, zNd, ones_ch)
    dvis, dei = _combine(dvp.reshape(2, N, DEGW), dep.reshape(2, N, DEGW))

    v2 = v_idx.reshape(P // CH, CH)
    e2 = e_idx.reshape(P // CH, CH)

    hs = _lin0(X, W0, b0.reshape(1, DH), dvis)
    ye = _gs64(hs.reshape(2 * N, 64), v2, e2, z64)
    yes = _escale64(ye.reshape(2, N, 64), dei)
    ho = _gs64(yes.reshape(2 * N, 64), e2, v2, z64)

    hs2 = _lin1(ho.reshape(2, N, 64), W1, b1.reshape(1, NCLS), dvis)
    ye2 = _gs32(hs2.reshape(2 * N, 32), v2, e2, z32)
    yes2 = _escale32(ye2.reshape(2, N, 32), dei)
    ho2 = _gs32(yes2.reshape(2 * N, 32), e2, v2, z32)

    return _final(ho2.reshape(2, N, 32), dvis)


# speedup vs baseline: 10.7312x; 1.3044x over previous
import functools

import jax
import jax.numpy as jnp
from jax import lax
from jax.experimental import pallas as pl
from jax.experimental.pallas import tpu as pltpu
from jax.experimental.pallas import tpu_sc as plsc

N = 10000
E = 10000
P = 320000
DIN = 128
DH = 128
NCLS = 64

NC = 2
NS = 16
CH = 125
DEGW = 16

f32 = jnp.float32


def _sc_mesh():
    return plsc.VectorSubcoreMesh(
        core_axis_name="c", subcore_axis_name="s", num_cores=NC, num_subcores=NS
    )


def _each_tile_rows(s, fn):
    @pl.when(s < NS - 1)
    def _():
        fn(pl.multiple_of(s * 624, 8), 624)

    @pl.when(s == NS - 1)
    def _():
        fn(624 * (NS - 1), N - 624 * (NS - 1))


NGD = 8
DGRP = P // NC // NS // CH // NGD


def _make_deg():
    @functools.partial(
        pl.kernel,
        out_type=[
            jax.ShapeDtypeStruct((2 * N, DEGW), f32),
            jax.ShapeDtypeStruct((2 * N, DEGW), f32),
        ],
        mesh=_sc_mesh(),
        scratch_types=[
            pltpu.VMEM((2, NGD, 2, CH), jnp.int32),
            pltpu.VMEM((CH, DEGW), f32),
            pltpu.SemaphoreType.DMA,
            pltpu.SemaphoreType.DMA,
            pltpu.VMEM_SHARED((N, DEGW), f32),
            pltpu.VMEM_SHARED((N, DEGW), f32),
        ],
        compiler_params=pltpu.CompilerParams(use_tc_tiling_on_sc=False),
    )
    def deg(idx_hbm, zeros_hbm, ones_hbm, dv_out, de_out,
            ib, ones_v, sem0, sem1, dv_s, de_s):
        c = lax.axis_index("c")
        s = lax.axis_index("s")
        sems = (sem0, sem1)
        pltpu.sync_copy(ones_hbm, ones_v)

        def zero_fn(off, sz):
            pltpu.sync_copy(zeros_hbm.at[pl.ds(off, sz)], dv_s.at[pl.ds(off, sz)])
            pltpu.sync_copy(zeros_hbm.at[pl.ds(off, sz)], de_s.at[pl.ds(off, sz)])

        _each_tile_rows(s, zero_fn)
        plsc.subcore_barrier()

        row0 = (c * NS + s) * (NGD * DGRP)

        def drain(p):
            for k in range(NGD):
                pltpu.make_async_copy(ones_v, dv_s.at[ib.at[p, k, 0]], sems[p]).wait()
                pltpu.make_async_copy(ones_v, de_s.at[ib.at[p, k, 1]], sems[p]).wait()

        def fire(g, p):
            @pl.when(g >= 2)
            def _():
                drain(p)

            pltpu.sync_copy(idx_hbm.at[pl.ds(row0 + g * NGD, NGD)], ib.at[p])
            for k in range(NGD):
                pltpu.async_copy(ones_v, dv_s.at[ib.at[p, k, 0]], sems[p], add=True)
                pltpu.async_copy(ones_v, de_s.at[ib.at[p, k, 1]], sems[p], add=True)

        fire(0, 0)
        fire(1, 1)

        @pl.loop(0, (DGRP - 2) // 2)
        def _(i):
            fire(2 * i + 2, 0)
            fire(2 * i + 3, 1)

        drain(0)
        drain(1)
        plsc.subcore_barrier()

        def out_fn(off, sz):
            dst = pl.multiple_of(c * N + off, 8)
            pltpu.sync_copy(dv_s.at[pl.ds(off, sz)], dv_out.at[pl.ds(dst, sz)])
            pltpu.sync_copy(de_s.at[pl.ds(off, sz)], de_out.at[pl.ds(dst, sz)])

        _each_tile_rows(s, out_fn)

    return deg


_deg = _make_deg()


CPT = P // NS // CH


def _make_gs(D, NG):
    @functools.partial(
        pl.kernel,
        out_type=jax.ShapeDtypeStruct((2 * N, D), f32),
        mesh=_sc_mesh(),
        scratch_types=[
            pltpu.VMEM((2, NG, 2, CH), jnp.int32),
            pltpu.VMEM((2, NG, CH, D), f32),
            pltpu.SemaphoreType.DMA,
            pltpu.SemaphoreType.DMA,
            pltpu.VMEM_SHARED((N, D), f32),
        ],
        compiler_params=pltpu.CompilerParams(use_tc_tiling_on_sc=False),
    )
    def gs(t0_hbm, t1_hbm, idx_hbm, zeros_hbm, o_hbm,
           ib, rows, sem0, sem1, acc):
        GRP = CPT // NG
        c = lax.axis_index("c")
        s = lax.axis_index("s")
        sems = (sem0, sem1)

        def zero_fn(off, sz):
            pltpu.sync_copy(zeros_hbm.at[pl.ds(off, sz)], acc.at[pl.ds(off, sz)])

        _each_tile_rows(s, zero_fn)
        plsc.subcore_barrier()

        row0 = s * CPT

        def fire(g, p):
            pltpu.sync_copy(idx_hbm.at[pl.ds(row0 + g * NG, NG)], ib.at[p])

            @pl.when(c == 0)
            def _():
                for k in range(NG):
                    pltpu.async_copy(t0_hbm.at[ib.at[p, k, 0]], rows.at[p, k], sems[p])

            @pl.when(c == 1)
            def _():
                for k in range(NG):
                    pltpu.async_copy(t1_hbm.at[ib.at[p, k, 0]], rows.at[p, k], sems[p])

        def consume(p):
            for k in range(NG):
                pltpu.make_async_copy(
                    t0_hbm.at[ib.at[p, k, 0]], rows.at[p, k], sems[p]
                ).wait()
            for k in range(NG):
                pltpu.sync_copy(rows.at[p, k], acc.at[ib.at[p, k, 1]], add=True)

        fire(0, 0)

        @pl.loop(0, (GRP + 1) // 2)
        def _(i):
            g = 2 * i

            @pl.when(g + 1 < GRP)
            def _():
                fire(g + 1, 1)

            consume(0)

            @pl.when(g + 2 < GRP)
            def _():
                fire(g + 2, 0)

            @pl.when(g + 1 < GRP)
            def _():
                consume(1)

        plsc.subcore_barrier()

        def out_fn(off, sz):
            dst = pl.multiple_of(c * N + off, 8)
            pltpu.sync_copy(acc.at[pl.ds(off, sz)], o_hbm.at[pl.ds(dst, sz)])

        _each_tile_rows(s, out_fn)

    return gs


_gs64 = _make_gs(64, 4)
_gs32 = _make_gs(32, 8)


def _combine_body(dvp, dep, dvis, dei):
    dv = dvp[0, :, 0:1] + dvp[1, :, 0:1]
    de = dep[0, :, 0:1] + dep[1, :, 0:1]
    dvis[...] = jnp.where(dv > 0, lax.rsqrt(dv), 0.0)
    dei[...] = jnp.where(de > 0, 1.0 / de, 0.0)


_combine = pl.pallas_call(
    _combine_body,
    grid=(10,),
    in_specs=[
        pl.BlockSpec((2, N // 10, DEGW), lambda i: (0, i, 0)),
        pl.BlockSpec((2, N // 10, DEGW), lambda i: (0, i, 0)),
    ],
    out_specs=[
        pl.BlockSpec((N // 10, 1), lambda i: (i, 0)),
        pl.BlockSpec((N // 10, 1), lambda i: (i, 0)),
    ],
    out_shape=[
        jax.ShapeDtypeStruct((N, 1), f32),
        jax.ShapeDtypeStruct((N, 1), f32),
    ],
)

_BR = 1000


def _lin0_body(x, w, b, dvis, o):
    h = jnp.dot(x[...], w[...], preferred_element_type=f32) + b[...]
    hs = h * dvis[...]
    o[0] = hs[:, :64]
    o[1] = hs[:, 64:]


_lin0 = pl.pallas_call(
    _lin0_body,
    grid=(N // _BR,),
    in_specs=[
        pl.BlockSpec((_BR, DIN), lambda i: (i, 0)),
        pl.BlockSpec((DIN, DH), lambda i: (0, 0)),
        pl.BlockSpec((1, DH), lambda i: (0, 0)),
        pl.BlockSpec((_BR, 1), lambda i: (i, 0)),
    ],
    out_specs=pl.BlockSpec((2, _BR, 64), lambda i: (0, i, 0)),
    out_shape=jax.ShapeDtypeStruct((2, N, 64), f32),
)


def _make_escale(D):
    def body(ye, dei, o):
        o[...] = ye[...] * dei[...][None]

    return pl.pallas_call(
        body,
        grid=(N // _BR,),
        in_specs=[
            pl.BlockSpec((2, _BR, D), lambda i: (0, i, 0)),
            pl.BlockSpec((_BR, 1), lambda i: (i, 0)),
        ],
        out_specs=pl.BlockSpec((2, _BR, D), lambda i: (0, i, 0)),
        out_shape=jax.ShapeDtypeStruct((2, N, D), f32),
    )


_escale64 = _make_escale(64)
_escale32 = _make_escale(32)


def _lin1_body(ho, w, b, dvis, o):
    sm = jnp.concatenate([ho[0], ho[1]], axis=1) * dvis[...]
    emb = jnp.maximum(sm, 0.0)
    h2 = jnp.dot(emb, w[...], preferred_element_type=f32) + b[...]
    hs2 = h2 * dvis[...]
    o[0] = hs2[:, :32]
    o[1] = hs2[:, 32:]


_lin1 = pl.pallas_call(
    _lin1_body,
    grid=(N // _BR,),
    in_specs=[
        pl.BlockSpec((2, _BR, 64), lambda i: (0, i, 0)),
        pl.BlockSpec((DH, NCLS), lambda i: (0, 0)),
        pl.BlockSpec((1, NCLS), lambda i: (0, 0)),
        pl.BlockSpec((_BR, 1), lambda i: (i, 0)),
    ],
    out_specs=pl.BlockSpec((2, _BR, 32), lambda i: (0, i, 0)),
    out_shape=jax.ShapeDtypeStruct((2, N, 32), f32),
)


def _final_body(ho, dvis, o):
    o[...] = jnp.concatenate([ho[0], ho[1]], axis=1) * dvis[...]


_final = pl.pallas_call(
    _final_body,
    grid=(N // _BR,),
    in_specs=[
        pl.BlockSpec((2, _BR, 32), lambda i: (0, i, 0)),
        pl.BlockSpec((_BR, 1), lambda i: (i, 0)),
    ],
    out_specs=pl.BlockSpec((_BR, NCLS), lambda i: (i, 0)),
    out_shape=jax.ShapeDtypeStruct((N, NCLS), f32),
)


def kernel(X, incidence, W0, b0, W1, b1):
    v_idx = incidence[0]
    e_idx = incidence[1]
    zNd = jnp.zeros((N, DEGW), f32)
    ones_ch = jnp.ones((CH, DEGW), f32)
    z64 = jnp.zeros((N, 64), f32)
    z32 = jnp.zeros((N, 32), f32)

    v2 = v_idx.reshape(P // CH, CH)
    e2 = e_idx.reshape(P // CH, CH)
    ive = jnp.stack([v2, e2], axis=1)
    iev = jnp.stack([e2, v2], axis=1)

    dvp, dep = _deg(ive, zNd, ones_ch)
    dvis, dei = _combine(dvp.reshape(2, N, DEGW), dep.reshape(2, N, DEGW))

    hs = _lin0(X, W0, b0.reshape(1, DH), dvis)
    ye = _gs64(hs[0], hs[1], ive, z64)
    yes = _escale64(ye.reshape(2, N, 64), dei)
    ho = _gs64(yes[0], yes[1], iev, z64)

    hs2 = _lin1(ho.reshape(2, N, 64), W1, b1.reshape(1, NCLS), dvis)
    ye2 = _gs32(hs2[0], hs2[1], ive, z32)
    yes2 = _escale32(ye2.reshape(2, N, 32), dei)
    ho2 = _gs32(yes2[0], yes2[1], iev, z32)

    return _final(ho2.reshape(2, N, 32), dvis)


# speedup vs baseline: 10.9947x; 1.0246x over previous
import functools

import jax
import jax.numpy as jnp
from jax import lax
from jax.experimental import pallas as pl
from jax.experimental.pallas import tpu as pltpu
from jax.experimental.pallas import tpu_sc as plsc

N = 10000
E = 10000
P = 320000
DIN = 128
DH = 128
NCLS = 64

NC = 2
NS = 16
CH = 125
DEGW = 16

f32 = jnp.float32


def _sc_mesh():
    return plsc.VectorSubcoreMesh(
        core_axis_name="c", subcore_axis_name="s", num_cores=NC, num_subcores=NS
    )


def _each_tile_rows(s, fn):
    @pl.when(s < NS - 1)
    def _():
        fn(pl.multiple_of(s * 624, 8), 624)

    @pl.when(s == NS - 1)
    def _():
        fn(624 * (NS - 1), N - 624 * (NS - 1))


NGD = 8
DGRP = P // NC // NS // CH // NGD


def _make_deg():
    @functools.partial(
        pl.kernel,
        out_type=[
            jax.ShapeDtypeStruct((2 * N, DEGW), f32),
            jax.ShapeDtypeStruct((2 * N, DEGW), f32),
        ],
        mesh=_sc_mesh(),
        scratch_types=[
            pltpu.VMEM((2, NGD, 2, CH), jnp.int32),
            pltpu.VMEM((CH, DEGW), f32),
            pltpu.SemaphoreType.DMA,
            pltpu.SemaphoreType.DMA,
            pltpu.VMEM_SHARED((N, DEGW), f32),
            pltpu.VMEM_SHARED((N, DEGW), f32),
        ],
        compiler_params=pltpu.CompilerParams(use_tc_tiling_on_sc=False),
    )
    def deg(idx_hbm, zeros_hbm, ones_hbm, dv_out, de_out,
            ib, ones_v, sem0, sem1, dv_s, de_s):
        c = lax.axis_index("c")
        s = lax.axis_index("s")
        sems = (sem0, sem1)
        pltpu.sync_copy(ones_hbm, ones_v)

        def zero_fn(off, sz):
            pltpu.sync_copy(zeros_hbm.at[pl.ds(off, sz)], dv_s.at[pl.ds(off, sz)])
            pltpu.sync_copy(zeros_hbm.at[pl.ds(off, sz)], de_s.at[pl.ds(off, sz)])

        _each_tile_rows(s, zero_fn)
        plsc.subcore_barrier()

        row0 = (c * NS + s) * (NGD * DGRP)

        def drain(p):
            for k in range(NGD):
                pltpu.make_async_copy(ones_v, dv_s.at[ib.at[p, k, 0]], sems[p]).wait()
                pltpu.make_async_copy(ones_v, de_s.at[ib.at[p, k, 1]], sems[p]).wait()

        def fire(g, p):
            @pl.when(g >= 2)
            def _():
                drain(p)

            pltpu.sync_copy(idx_hbm.at[pl.ds(row0 + g * NGD, NGD)], ib.at[p])
            for k in range(NGD):
                pltpu.async_copy(ones_v, dv_s.at[ib.at[p, k, 0]], sems[p], add=True)
                pltpu.async_copy(ones_v, de_s.at[ib.at[p, k, 1]], sems[p], add=True)

        fire(0, 0)
        fire(1, 1)

        @pl.loop(0, (DGRP - 2) // 2)
        def _(i):
            fire(2 * i + 2, 0)
            fire(2 * i + 3, 1)

        drain(0)
        drain(1)
        plsc.subcore_barrier()

        def out_fn(off, sz):
            dst = pl.multiple_of(c * N + off, 8)
            pltpu.sync_copy(dv_s.at[pl.ds(off, sz)], dv_out.at[pl.ds(dst, sz)])
            pltpu.sync_copy(de_s.at[pl.ds(off, sz)], de_out.at[pl.ds(dst, sz)])

        _each_tile_rows(s, out_fn)

    return deg


_deg = _make_deg()


CPT = P // NS // CH


def _make_gs(D, NG):
    @functools.partial(
        pl.kernel,
        out_type=jax.ShapeDtypeStruct((2 * N, D), f32),
        mesh=_sc_mesh(),
        scratch_types=[
            pltpu.VMEM((3, NG, 2, CH), jnp.int32),
            pltpu.VMEM((3, NG, CH, D), f32),
            pltpu.SemaphoreType.DMA,
            pltpu.SemaphoreType.DMA,
            pltpu.SemaphoreType.DMA,
            pltpu.SemaphoreType.DMA,
            pltpu.SemaphoreType.DMA,
            pltpu.SemaphoreType.DMA,
            pltpu.VMEM_SHARED((N, D), f32),
        ],
        compiler_params=pltpu.CompilerParams(use_tc_tiling_on_sc=False),
    )
    def gs(t0_hbm, t1_hbm, idx_hbm, zeros_hbm, o_hbm,
           ib, rows, g0, g1, g2, s0, s1, s2, acc):
        GRP = CPT // NG
        NT = GRP + 1
        ITER = (NT + 2) // 3
        c = lax.axis_index("c")
        s = lax.axis_index("s")
        gsems = (g0, g1, g2)
        ssems = (s0, s1, s2)

        def zero_fn(off, sz):
            pltpu.sync_copy(zeros_hbm.at[pl.ds(off, sz)], acc.at[pl.ds(off, sz)])

        _each_tile_rows(s, zero_fn)
        plsc.subcore_barrier()

        row0 = s * CPT

        def fire(g, p):
            pltpu.sync_copy(idx_hbm.at[pl.ds(row0 + g * NG, NG)], ib.at[p])

            @pl.when(c == 0)
            def _():
                for k in range(NG):
                    pltpu.async_copy(t0_hbm.at[ib.at[p, k, 0]], rows.at[p, k], gsems[p])

            @pl.when(c == 1)
            def _():
                for k in range(NG):
                    pltpu.async_copy(t1_hbm.at[ib.at[p, k, 0]], rows.at[p, k], gsems[p])

        def consume(p):
            for k in range(NG):
                pltpu.make_async_copy(
                    t0_hbm.at[ib.at[p, k, 0]], rows.at[p, k], gsems[p]
                ).wait()
            for k in range(NG):
                pltpu.async_copy(rows.at[p, k], acc.at[ib.at[p, k, 1]], ssems[p], add=True)

        def sdrain(p):
            for k in range(NG):
                pltpu.make_async_copy(
                    rows.at[p, k], acc.at[ib.at[p, k, 1]], ssems[p]
                ).wait()

        @pl.loop(0, ITER)
        def _(i):
            for u in range(3):
                t = 3 * i + u

                @pl.when(t >= 3)
                def _():
                    sdrain(u)

                @pl.when(t < GRP)
                def _():
                    fire(t, u)

                @pl.when((t >= 1) & (t <= GRP))
                def _():
                    consume((u + 2) % 3)

        for g in range(3 * ITER - 3, GRP):
            sdrain(g % 3)

        plsc.subcore_barrier()

        def out_fn(off, sz):
            dst = pl.multiple_of(c * N + off, 8)
            pltpu.sync_copy(acc.at[pl.ds(off, sz)], o_hbm.at[pl.ds(dst, sz)])

        _each_tile_rows(s, out_fn)

    return gs


_gs64 = _make_gs(64, 2)
_gs32 = _make_gs(32, 4)


def _combine_body(dvp, dep, dvis, dei):
    dv = dvp[0, :, 0:1] + dvp[1, :, 0:1]
    de = dep[0, :, 0:1] + dep[1, :, 0:1]
    dvis[...] = jnp.where(dv > 0, lax.rsqrt(dv), 0.0)
    dei[...] = jnp.where(de > 0, 1.0 / de, 0.0)


_combine = pl.pallas_call(
    _combine_body,
    grid=(10,),
    in_specs=[
        pl.BlockSpec((2, N // 10, DEGW), lambda i: (0, i, 0)),
        pl.BlockSpec((2, N // 10, DEGW), lambda i: (0, i, 0)),
    ],
    out_specs=[
        pl.BlockSpec((N // 10, 1), lambda i: (i, 0)),
        pl.BlockSpec((N // 10, 1), lambda i: (i, 0)),
    ],
    out_shape=[
        jax.ShapeDtypeStruct((N, 1), f32),
        jax.ShapeDtypeStruct((N, 1), f32),
    ],
)

_BR = 1000


def _lin0_body(x, w, b, dvis, o):
    h = jnp.dot(x[...], w[...], preferred_element_type=f32) + b[...]
    hs = h * dvis[...]
    o[0] = hs[:, :64]
    o[1] = hs[:, 64:]


_lin0 = pl.pallas_call(
    _lin0_body,
    grid=(N // _BR,),
    in_specs=[
        pl.BlockSpec((_BR, DIN), lambda i: (i, 0)),
        pl.BlockSpec((DIN, DH), lambda i: (0, 0)),
        pl.BlockSpec((1, DH), lambda i: (0, 0)),
        pl.BlockSpec((_BR, 1), lambda i: (i, 0)),
    ],
    out_specs=pl.BlockSpec((2, _BR, 64), lambda i: (0, i, 0)),
    out_shape=jax.ShapeDtypeStruct((2, N, 64), f32),
)


def _make_escale(D):
    def body(ye, dei, o):
        o[...] = ye[...] * dei[...][None]

    return pl.pallas_call(
        body,
        grid=(N // _BR,),
        in_specs=[
            pl.BlockSpec((2, _BR, D), lambda i: (0, i, 0)),
            pl.BlockSpec((_BR, 1), lambda i: (i, 0)),
        ],
        out_specs=pl.BlockSpec((2, _BR, D), lambda i: (0, i, 0)),
        out_shape=jax.ShapeDtypeStruct((2, N, D), f32),
    )


_escale64 = _make_escale(64)
_escale32 = _make_escale(32)


def _lin1_body(ho, w, b, dvis, o):
    sm = jnp.concatenate([ho[0], ho[1]], axis=1) * dvis[...]
    emb = jnp.maximum(sm, 0.0)
    h2 = jnp.dot(emb, w[...], preferred_element_type=f32) + b[...]
    hs2 = h2 * dvis[...]
    o[0] = hs2[:, :32]
    o[1] = hs2[:, 32:]


_lin1 = pl.pallas_call(
    _lin1_body,
    grid=(N // _BR,),
    in_specs=[
        pl.BlockSpec((2, _BR, 64), lambda i: (0, i, 0)),
        pl.BlockSpec((DH, NCLS), lambda i: (0, 0)),
        pl.BlockSpec((1, NCLS), lambda i: (0, 0)),
        pl.BlockSpec((_BR, 1), lambda i: (i, 0)),
    ],
    out_specs=pl.BlockSpec((2, _BR, 32), lambda i: (0, i, 0)),
    out_shape=jax.ShapeDtypeStruct((2, N, 32), f32),
)


def _final_body(ho, dvis, o):
    o[...] = jnp.concatenate([ho[0], ho[1]], axis=1) * dvis[...]


_final = pl.pallas_call(
    _final_body,
    grid=(N // _BR,),
    in_specs=[
        pl.BlockSpec((2, _BR, 32), lambda i: (0, i, 0)),
        pl.BlockSpec((_BR, 1), lambda i: (i, 0)),
    ],
    out_specs=pl.BlockSpec((_BR, NCLS), lambda i: (i, 0)),
    out_shape=jax.ShapeDtypeStruct((N, NCLS), f32),
)


def kernel(X, incidence, W0, b0, W1, b1):
    v_idx = incidence[0]
    e_idx = incidence[1]
    zNd = jnp.zeros((N, DEGW), f32)
    ones_ch = jnp.ones((CH, DEGW), f32)
    z64 = jnp.zeros((N, 64), f32)
    z32 = jnp.zeros((N, 32), f32)

    v2 = v_idx.reshape(P // CH, CH)
    e2 = e_idx.reshape(P // CH, CH)
    ive = jnp.stack([v2, e2], axis=1)
    iev = jnp.stack([e2, v2], axis=1)

    dvp, dep = _deg(ive, zNd, ones_ch)
    dvis, dei = _combine(dvp.reshape(2, N, DEGW), dep.reshape(2, N, DEGW))

    hs = _lin0(X, W0, b0.reshape(1, DH), dvis)
    ye = _gs64(hs[0], hs[1], ive, z64)
    yes = _escale64(ye.reshape(2, N, 64), dei)
    ho = _gs64(yes[0], yes[1], iev, z64)

    hs2 = _lin1(ho.reshape(2, N, 64), W1, b1.reshape(1, NCLS), dvis)
    ye2 = _gs32(hs2[0], hs2[1], ive, z32)
    yes2 = _escale32(ye2.reshape(2, N, 32), dei)
    ho2 = _gs32(yes2[0], yes2[1], iev, z32)

    return _final(ho2.reshape(2, N, 32), dvis)


# speedup vs baseline: 12.3068x; 1.1193x over previous
import functools

import jax
import jax.numpy as jnp
from jax import lax
from jax.experimental import pallas as pl
from jax.experimental.pallas import tpu as pltpu
from jax.experimental.pallas import tpu_sc as plsc

N = 10000
E = 10000
P = 320000
DIN = 128
DH = 128
NCLS = 64

NC = 2
NS = 16
CH = 125
DEGW = 16

f32 = jnp.float32


def _sc_mesh():
    return plsc.VectorSubcoreMesh(
        core_axis_name="c", subcore_axis_name="s", num_cores=NC, num_subcores=NS
    )


def _each_tile_rows(s, fn):
    @pl.when(s < NS - 1)
    def _():
        fn(pl.multiple_of(s * 624, 8), 624)

    @pl.when(s == NS - 1)
    def _():
        fn(624 * (NS - 1), N - 624 * (NS - 1))


NGD = 8
DGRP = P // NC // NS // CH // NGD


def _make_deg():
    @functools.partial(
        pl.kernel,
        out_type=[
            jax.ShapeDtypeStruct((2 * N, DEGW), f32),
            jax.ShapeDtypeStruct((2 * N, DEGW), f32),
        ],
        mesh=_sc_mesh(),
        scratch_types=[
            pltpu.VMEM((2, NGD, 2, CH), jnp.int32),
            pltpu.VMEM((CH, DEGW), f32),
            pltpu.SemaphoreType.DMA,
            pltpu.SemaphoreType.DMA,
            pltpu.VMEM_SHARED((N, DEGW), f32),
            pltpu.VMEM_SHARED((N, DEGW), f32),
        ],
        compiler_params=pltpu.CompilerParams(use_tc_tiling_on_sc=False),
    )
    def deg(idx_hbm, zeros_hbm, ones_hbm, dv_out, de_out,
            ib, ones_v, sem0, sem1, dv_s, de_s):
        c = lax.axis_index("c")
        s = lax.axis_index("s")
        sems = (sem0, sem1)
        pltpu.sync_copy(ones_hbm, ones_v)

        def zero_fn(off, sz):
            pltpu.sync_copy(zeros_hbm.at[pl.ds(off, sz)], dv_s.at[pl.ds(off, sz)])
            pltpu.sync_copy(zeros_hbm.at[pl.ds(off, sz)], de_s.at[pl.ds(off, sz)])

        _each_tile_rows(s, zero_fn)
        plsc.subcore_barrier()

        row0 = (c * NS + s) * (NGD * DGRP)

        def drain(p):
            for k in range(NGD):
                pltpu.make_async_copy(ones_v, dv_s.at[ib.at[p, k, 0]], sems[p]).wait()
                pltpu.make_async_copy(ones_v, de_s.at[ib.at[p, k, 1]], sems[p]).wait()

        def fire(g, p):
            @pl.when(g >= 2)
            def _():
                drain(p)

            pltpu.sync_copy(idx_hbm.at[pl.ds(row0 + g * NGD, NGD)], ib.at[p])
            for k in range(NGD):
                pltpu.async_copy(ones_v, dv_s.at[ib.at[p, k, 0]], sems[p], add=True)
                pltpu.async_copy(ones_v, de_s.at[ib.at[p, k, 1]], sems[p], add=True)

        fire(0, 0)
        fire(1, 1)

        @pl.loop(0, (DGRP - 2) // 2)
        def _(i):
            fire(2 * i + 2, 0)
            fire(2 * i + 3, 1)

        drain(0)
        drain(1)
        plsc.subcore_barrier()

        def out_fn(off, sz):
            dst = pl.multiple_of(c * N + off, 8)
            pltpu.sync_copy(dv_s.at[pl.ds(off, sz)], dv_out.at[pl.ds(dst, sz)])
            pltpu.sync_copy(de_s.at[pl.ds(off, sz)], de_out.at[pl.ds(dst, sz)])

        _each_tile_rows(s, out_fn)

    return deg


_deg = _make_deg()


CPT = P // NS // CH


def _make_gs(D, NG):
    @functools.partial(
        pl.kernel,
        out_type=jax.ShapeDtypeStruct((2 * N, D), f32),
        mesh=_sc_mesh(),
        scratch_types=[
            pltpu.VMEM((3, NG, 2, CH), jnp.int32),
            pltpu.VMEM((3, NG, CH, D), f32),
            pltpu.SemaphoreType.DMA,
            pltpu.SemaphoreType.DMA,
            pltpu.SemaphoreType.DMA,
            pltpu.SemaphoreType.DMA,
            pltpu.SemaphoreType.DMA,
            pltpu.SemaphoreType.DMA,
            pltpu.VMEM_SHARED((N, D), f32),
        ],
        compiler_params=pltpu.CompilerParams(use_tc_tiling_on_sc=False),
    )
    def gs(t0_hbm, t1_hbm, idx_hbm, zeros_hbm, o_hbm,
           ib, rows, g0, g1, g2, s0, s1, s2, acc):
        GRP = CPT // NG
        NT = GRP + 1
        ITER = (NT + 2) // 3
        c = lax.axis_index("c")
        s = lax.axis_index("s")
        gsems = (g0, g1, g2)
        ssems = (s0, s1, s2)

        def zero_fn(off, sz):
            pltpu.sync_copy(zeros_hbm.at[pl.ds(off, sz)], acc.at[pl.ds(off, sz)])

        _each_tile_rows(s, zero_fn)
        plsc.subcore_barrier()

        row0 = s * CPT

        def fire(g, p):
            pltpu.sync_copy(idx_hbm.at[pl.ds(row0 + g * NG, NG)], ib.at[p])

            @pl.when(c == 0)
            def _():
                for k in range(NG):
                    pltpu.async_copy(t0_hbm.at[ib.at[p, k, 0]], rows.at[p, k], gsems[p])

            @pl.when(c == 1)
            def _():
                for k in range(NG):
                    pltpu.async_copy(t1_hbm.at[ib.at[p, k, 0]], rows.at[p, k], gsems[p])

        def consume(p):
            for k in range(NG):
                pltpu.make_async_copy(
                    t0_hbm.at[ib.at[p, k, 0]], rows.at[p, k], gsems[p]
                ).wait()
            for k in range(NG):
                pltpu.async_copy(rows.at[p, k], acc.at[ib.at[p, k, 1]], ssems[p], add=True)

        def sdrain(p):
            for k in range(NG):
                pltpu.make_async_copy(
                    rows.at[p, k], acc.at[ib.at[p, k, 1]], ssems[p]
                ).wait()

        @pl.loop(0, ITER)
        def _(i):
            for u in range(3):
                t = 3 * i + u

                @pl.when(t >= 3)
                def _():
                    sdrain(u)

                @pl.when(t < GRP)
                def _():
                    fire(t, u)

                @pl.when((t >= 1) & (t <= GRP))
                def _():
                    consume((u + 2) % 3)

        for g in range(3 * ITER - 3, GRP):
            sdrain(g % 3)

        plsc.subcore_barrier()

        def out_fn(off, sz):
            dst = pl.multiple_of(c * N + off, 8)
            pltpu.sync_copy(acc.at[pl.ds(off, sz)], o_hbm.at[pl.ds(dst, sz)])

        _each_tile_rows(s, out_fn)

    return gs


_gs64 = _make_gs(64, 2)
_gs32 = _make_gs(32, 4)


def _make_smooth(D, NG):
    RS = 48

    @functools.partial(
        pl.kernel,
        out_type=[
            jax.ShapeDtypeStruct((2 * N, D), f32),
            jax.ShapeDtypeStruct((N, D), f32),
            jax.ShapeDtypeStruct((N, D), f32),
        ],
        mesh=_sc_mesh(),
        scratch_types=[
            pltpu.VMEM((3, NG, 2, CH), jnp.int32),
            pltpu.VMEM((3, NG, CH, D), f32),
            pltpu.VMEM((80, D), f32),
            pltpu.VMEM((640,), f32),
            pltpu.SemaphoreType.DMA,
            pltpu.SemaphoreType.DMA,
            pltpu.SemaphoreType.DMA,
            pltpu.SemaphoreType.DMA,
            pltpu.SemaphoreType.DMA,
            pltpu.SemaphoreType.DMA,
            pltpu.VMEM_SHARED((N, D), f32),
        ],
        compiler_params=pltpu.CompilerParams(use_tc_tiling_on_sc=False),
    )
    def smooth(t0_hbm, t1_hbm, idx_a_hbm, idx_b_hbm, dei_hbm, zeros_hbm,
               o_hbm, y0_hbm, y1_hbm,
               ib, rows, srow, dbuf, g0, g1, g2, s0, s1, s2, acc):
        GRP = CPT // NG
        NT = GRP + 1
        ITER = (NT + 2) // 3
        c = lax.axis_index("c")
        s = lax.axis_index("s")
        gsems = (g0, g1, g2)
        ssems = (s0, s1, s2)
        row0 = s * CPT

        def zero_fn(off, sz):
            pltpu.sync_copy(zeros_hbm.at[pl.ds(off, sz)], acc.at[pl.ds(off, sz)])

        def seg_pass(idx_hbm, ta_hbm, tb_hbm):
            def fire(g, p):
                pltpu.sync_copy(idx_hbm.at[pl.ds(row0 + g * NG, NG)], ib.at[p])

                @pl.when(c == 0)
                def _():
                    for k in range(NG):
                        pltpu.async_copy(ta_hbm.at[ib.at[p, k, 0]], rows.at[p, k], gsems[p])

                @pl.when(c == 1)
                def _():
                    for k in range(NG):
                        pltpu.async_copy(tb_hbm.at[ib.at[p, k, 0]], rows.at[p, k], gsems[p])

            def consume(p):
                for k in range(NG):
                    pltpu.make_async_copy(
                        ta_hbm.at[ib.at[p, k, 0]], rows.at[p, k], gsems[p]
                    ).wait()
                for k in range(NG):
                    pltpu.async_copy(rows.at[p, k], acc.at[ib.at[p, k, 1]], ssems[p], add=True)

            def sdrain(p):
                for k in range(NG):
                    pltpu.make_async_copy(
                        rows.at[p, k], acc.at[ib.at[p, k, 1]], ssems[p]
                    ).wait()

            @pl.loop(0, ITER)
            def _(i):
                for u in range(3):
                    t = 3 * i + u

                    @pl.when(t >= 3)
                    def _():
                        sdrain(u)

                    @pl.when(t < GRP)
                    def _():
                        fire(t, u)

                    @pl.when((t >= 1) & (t <= GRP))
                    def _():
                        consume((u + 2) % 3)

            for g in range(3 * ITER - 3, GRP):
                sdrain(g % 3)

        def scale_to(y_hbm, off, sz, rs):
            pltpu.sync_copy(dei_hbm.at[pl.ds(off, sz)], dbuf.at[pl.ds(0, sz)])

            @pl.loop(0, sz // rs)
            def _(bi):
                r0 = off + bi * rs
                pltpu.sync_copy(acc.at[pl.ds(r0, rs)], srow.at[pl.ds(0, rs)])
                for q in range(rs // 16):
                    d16 = dbuf[pl.ds(bi * rs + 16 * q, 16)]
                    for r in range(16):
                        d = d16[r]
                        rr = 16 * q + r
                        for jj in range(D // 16):
                            srow[rr, pl.ds(16 * jj, 16)] = (
                                srow[rr, pl.ds(16 * jj, 16)] * d
                            )
                pltpu.sync_copy(srow.at[pl.ds(0, rs)], y_hbm.at[pl.ds(r0, rs)])

        def scale_fn(off, sz):
            rs = RS if sz == 624 else 80

            @pl.when(c == 0)
            def _():
                scale_to(y0_hbm, off, sz, rs)

            @pl.when(c == 1)
            def _():
                scale_to(y1_hbm, off, sz, rs)

        _each_tile_rows(s, zero_fn)
        plsc.subcore_barrier()
        seg_pass(idx_a_hbm, t0_hbm, t1_hbm)
        plsc.subcore_barrier()

        _each_tile_rows(s, scale_fn)
        plsc.subcore_barrier()
        _each_tile_rows(s, zero_fn)
        plsc.subcore_barrier()

        seg_pass(idx_b_hbm, y0_hbm, y1_hbm)
        plsc.subcore_barrier()

        def out_fn(off, sz):
            dst = pl.multiple_of(c * N + off, 8)
            pltpu.sync_copy(acc.at[pl.ds(off, sz)], o_hbm.at[pl.ds(dst, sz)])

        _each_tile_rows(s, out_fn)

    return smooth


_smooth64 = _make_smooth(64, 2)
_smooth32 = _make_smooth(32, 4)


def _combine_body(dvp, dep, dvis, dei):
    dv = dvp[0, :, 0:1] + dvp[1, :, 0:1]
    de = dep[0, :, 0:1] + dep[1, :, 0:1]
    dvis[...] = jnp.where(dv > 0, lax.rsqrt(dv), 0.0)
    dei[...] = jnp.where(de > 0, 1.0 / de, 0.0)


_combine = pl.pallas_call(
    _combine_body,
    grid=(10,),
    in_specs=[
        pl.BlockSpec((2, N // 10, DEGW), lambda i: (0, i, 0)),
        pl.BlockSpec((2, N // 10, DEGW), lambda i: (0, i, 0)),
    ],
    out_specs=[
        pl.BlockSpec((N // 10, 1), lambda i: (i, 0)),
        pl.BlockSpec((N // 10, 1), lambda i: (i, 0)),
    ],
    out_shape=[
        jax.ShapeDtypeStruct((N, 1), f32),
        jax.ShapeDtypeStruct((N, 1), f32),
    ],
)

_BR = 1000


def _lin0_body(x, w, b, dvis, o):
    h = jnp.dot(x[...], w[...], preferred_element_type=f32) + b[...]
    hs = h * dvis[...]
    o[0] = hs[:, :64]
    o[1] = hs[:, 64:]


_lin0 = pl.pallas_call(
    _lin0_body,
    grid=(N // _BR,),
    in_specs=[
        pl.BlockSpec((_BR, DIN), lambda i: (i, 0)),
        pl.BlockSpec((DIN, DH), lambda i: (0, 0)),
        pl.BlockSpec((1, DH), lambda i: (0, 0)),
        pl.BlockSpec((_BR, 1), lambda i: (i, 0)),
    ],
    out_specs=pl.BlockSpec((2, _BR, 64), lambda i: (0, i, 0)),
    out_shape=jax.ShapeDtypeStruct((2, N, 64), f32),
)


def _make_escale(D):
    def body(ye, dei, o):
        o[...] = ye[...] * dei[...][None]

    return pl.pallas_call(
        body,
        grid=(N // _BR,),
        in_specs=[
            pl.BlockSpec((2, _BR, D), lambda i: (0, i, 0)),
            pl.BlockSpec((_BR, 1), lambda i: (i, 0)),
        ],
        out_specs=pl.BlockSpec((2, _BR, D), lambda i: (0, i, 0)),
        out_shape=jax.ShapeDtypeStruct((2, N, D), f32),
    )


_escale64 = _make_escale(64)
_escale32 = _make_escale(32)


def _lin1_body(ho, w, b, dvis, o):
    sm = jnp.concatenate([ho[0], ho[1]], axis=1) * dvis[...]
    emb = jnp.maximum(sm, 0.0)
    h2 = jnp.dot(emb, w[...], preferred_element_type=f32) + b[...]
    hs2 = h2 * dvis[...]
    o[0] = hs2[:, :32]
    o[1] = hs2[:, 32:]


_lin1 = pl.pallas_call(
    _lin1_body,
    grid=(N // _BR,),
    in_specs=[
        pl.BlockSpec((2, _BR, 64), lambda i: (0, i, 0)),
        pl.BlockSpec((DH, NCLS), lambda i: (0, 0)),
        pl.BlockSpec((1, NCLS), lambda i: (0, 0)),
        pl.BlockSpec((_BR, 1), lambda i: (i, 0)),
    ],
    out_specs=pl.BlockSpec((2, _BR, 32), lambda i: (0, i, 0)),
    out_shape=jax.ShapeDtypeStruct((2, N, 32), f32),
)


def _final_body(ho, dvis, o):
    o[...] = jnp.concatenate([ho[0], ho[1]], axis=1) * dvis[...]


_final = pl.pallas_call(
    _final_body,
    grid=(N // _BR,),
    in_specs=[
        pl.BlockSpec((2, _BR, 32), lambda i: (0, i, 0)),
        pl.BlockSpec((_BR, 1), lambda i: (i, 0)),
    ],
    out_specs=pl.BlockSpec((_BR, NCLS), lambda i: (i, 0)),
    out_shape=jax.ShapeDtypeStruct((N, NCLS), f32),
)


def kernel(X, incidence, W0, b0, W1, b1):
    v_idx = incidence[0]
    e_idx = incidence[1]
    zNd = jnp.zeros((N, DEGW), f32)
    ones_ch = jnp.ones((CH, DEGW), f32)
    z64 = jnp.zeros((N, 64), f32)
    z32 = jnp.zeros((N, 32), f32)

    v2 = v_idx.reshape(P // CH, CH)
    e2 = e_idx.reshape(P // CH, CH)
    ive = jnp.stack([v2, e2], axis=1)
    iev = jnp.stack([e2, v2], axis=1)

    dvp, dep = _deg(ive, zNd, ones_ch)
    dvis, dei = _combine(dvp.reshape(2, N, DEGW), dep.reshape(2, N, DEGW))

    dei_flat = dei.reshape(N)

    hs = _lin0(X, W0, b0.reshape(1, DH), dvis)
    ho, _, _ = _smooth64(hs[0], hs[1], ive, iev, dei_flat, z64)

    hs2 = _lin1(ho.reshape(2, N, 64), W1, b1.reshape(1, NCLS), dvis)
    ho2, _, _ = _smooth32(hs2[0], hs2[1], ive, iev, dei_flat, z32)

    return _final(ho2.reshape(2, N, 32), dvis)


# speedup vs baseline: 12.4616x; 1.0126x over previous
import functools

import jax
import jax.numpy as jnp
from jax import lax
from jax.experimental import pallas as pl
from jax.experimental.pallas import tpu as pltpu
from jax.experimental.pallas import tpu_sc as plsc

N = 10000
E = 10000
P = 320000
DIN = 128
DH = 128
NCLS = 64

NC = 2
NS = 16
CH = 125
DEGW = 16

f32 = jnp.float32


def _sc_mesh():
    return plsc.VectorSubcoreMesh(
        core_axis_name="c", subcore_axis_name="s", num_cores=NC, num_subcores=NS
    )


def _each_tile_rows(s, fn):
    @pl.when(s < NS - 1)
    def _():
        fn(pl.multiple_of(s * 624, 8), 624)

    @pl.when(s == NS - 1)
    def _():
        fn(624 * (NS - 1), N - 624 * (NS - 1))


NGD = 8
DGRP = P // NC // NS // CH // NGD


def _make_deg():
    @functools.partial(
        pl.kernel,
        out_type=[
            jax.ShapeDtypeStruct((2 * N, DEGW), f32),
            jax.ShapeDtypeStruct((2 * N, DEGW), f32),
        ],
        mesh=_sc_mesh(),
        scratch_types=[
            pltpu.VMEM((2, NGD, 2, CH), jnp.int32),
            pltpu.VMEM((CH, DEGW), f32),
            pltpu.SemaphoreType.DMA,
            pltpu.SemaphoreType.DMA,
            pltpu.VMEM_SHARED((N, DEGW), f32),
            pltpu.VMEM_SHARED((N, DEGW), f32),
        ],
        compiler_params=pltpu.CompilerParams(use_tc_tiling_on_sc=False),
    )
    def deg(idx_hbm, zeros_hbm, ones_hbm, dv_out, de_out,
            ib, ones_v, sem0, sem1, dv_s, de_s):
        c = lax.axis_index("c")
        s = lax.axis_index("s")
        sems = (sem0, sem1)
        pltpu.sync_copy(ones_hbm, ones_v)

        def zero_fn(off, sz):
            pltpu.sync_copy(zeros_hbm.at[pl.ds(off, sz)], dv_s.at[pl.ds(off, sz)])
            pltpu.sync_copy(zeros_hbm.at[pl.ds(off, sz)], de_s.at[pl.ds(off, sz)])

        _each_tile_rows(s, zero_fn)
        plsc.subcore_barrier()

        row0 = (c * NS + s) * (NGD * DGRP)

        def drain(p):
            for k in range(NGD):
                pltpu.make_async_copy(ones_v, dv_s.at[ib.at[p, k, 0]], sems[p]).wait()
                pltpu.make_async_copy(ones_v, de_s.at[ib.at[p, k, 1]], sems[p]).wait()

        def fire(g, p):
            @pl.when(g >= 2)
            def _():
                drain(p)

            pltpu.sync_copy(idx_hbm.at[pl.ds(row0 + g * NGD, NGD)], ib.at[p])
            for k in range(NGD):
                pltpu.async_copy(ones_v, dv_s.at[ib.at[p, k, 0]], sems[p], add=True)
                pltpu.async_copy(ones_v, de_s.at[ib.at[p, k, 1]], sems[p], add=True)

        fire(0, 0)
        fire(1, 1)

        @pl.loop(0, (DGRP - 2) // 2)
        def _(i):
            fire(2 * i + 2, 0)
            fire(2 * i + 3, 1)

        drain(0)
        drain(1)
        plsc.subcore_barrier()

        def out_fn(off, sz):
            dst = pl.multiple_of(c * N + off, 8)
            pltpu.sync_copy(dv_s.at[pl.ds(off, sz)], dv_out.at[pl.ds(dst, sz)])
            pltpu.sync_copy(de_s.at[pl.ds(off, sz)], de_out.at[pl.ds(dst, sz)])

        _each_tile_rows(s, out_fn)

    return deg


_deg = _make_deg()


CPT = P // NS // CH


def _make_smooth(D, NG):
    RS = 48

    @functools.partial(
        pl.kernel,
        out_type=[
            jax.ShapeDtypeStruct((2 * N, D), f32),
            jax.ShapeDtypeStruct((N, D), f32),
            jax.ShapeDtypeStruct((N, D), f32),
        ],
        mesh=_sc_mesh(),
        scratch_types=[
            pltpu.VMEM((3, NG, 2, CH), jnp.int32),
            pltpu.VMEM((3, NG, CH, D), f32),
            pltpu.VMEM((80, D), f32),
            pltpu.VMEM((640,), f32),
            pltpu.SemaphoreType.DMA,
            pltpu.SemaphoreType.DMA,
            pltpu.SemaphoreType.DMA,
            pltpu.SemaphoreType.DMA,
            pltpu.SemaphoreType.DMA,
            pltpu.SemaphoreType.DMA,
            pltpu.VMEM_SHARED((N, D), f32),
        ],
        compiler_params=pltpu.CompilerParams(use_tc_tiling_on_sc=False),
    )
    def smooth(t0_hbm, t1_hbm, idx_a_hbm, idx_b_hbm, dei_hbm, zeros_hbm,
               o_hbm, y0_hbm, y1_hbm,
               ib, rows, srow, dbuf, g0, g1, g2, s0, s1, s2, acc):
        GRP = CPT // NG
        NT = GRP + 1
        ITER = (NT + 2) // 3
        c = lax.axis_index("c")
        s = lax.axis_index("s")
        gsems = (g0, g1, g2)
        ssems = (s0, s1, s2)
        row0 = s * CPT

        def zero_fn(off, sz):
            pltpu.sync_copy(zeros_hbm.at[pl.ds(off, sz)], acc.at[pl.ds(off, sz)])

        def seg_pass(idx_hbm, ta_hbm, tb_hbm):
            def fire(g, p):
                pltpu.sync_copy(idx_hbm.at[pl.ds(row0 + g * NG, NG)], ib.at[p])

                @pl.when(c == 0)
                def _():
                    for k in range(NG):
                        pltpu.async_copy(ta_hbm.at[ib.at[p, k, 0]], rows.at[p, k], gsems[p])

                @pl.when(c == 1)
                def _():
                    for k in range(NG):
                        pltpu.async_copy(tb_hbm.at[ib.at[p, k, 0]], rows.at[p, k], gsems[p])

            def consume(p):
                for k in range(NG):
                    pltpu.make_async_copy(
                        ta_hbm.at[ib.at[p, k, 0]], rows.at[p, k], gsems[p]
                    ).wait()
                for k in range(NG):
                    pltpu.async_copy(rows.at[p, k], acc.at[ib.at[p, k, 1]], ssems[p], add=True)

            def sdrain(p):
                for k in range(NG):
                    pltpu.make_async_copy(
                        rows.at[p, k], acc.at[ib.at[p, k, 1]], ssems[p]
                    ).wait()

            @pl.loop(0, ITER)
            def _(i):
                for u in range(3):
                    t = 3 * i + u

                    @pl.when(t >= 3)
                    def _():
                        sdrain(u)

                    @pl.when(t < GRP)
                    def _():
                        fire(t, u)

                    @pl.when((t >= 1) & (t <= GRP))
                    def _():
                        consume((u + 2) % 3)

            for g in range(3 * ITER - 3, GRP):
                sdrain(g % 3)

        def scale_to(y_hbm, off, sz, rs):
            pltpu.sync_copy(dei_hbm.at[pl.ds(off, sz)], dbuf.at[pl.ds(0, sz)])

            @pl.loop(0, sz // rs)
            def _(bi):
                r0 = off + bi * rs
                pltpu.sync_copy(acc.at[pl.ds(r0, rs)], srow.at[pl.ds(0, rs)])
                for q in range(rs // 16):
                    d16 = dbuf[pl.ds(bi * rs + 16 * q, 16)]
                    for r in range(16):
                        d = d16[r]
                        rr = 16 * q + r
                        for jj in range(D // 16):
                            srow[rr, pl.ds(16 * jj, 16)] = (
                                srow[rr, pl.ds(16 * jj, 16)] * d
                            )
                pltpu.sync_copy(srow.at[pl.ds(0, rs)], y_hbm.at[pl.ds(r0, rs)])

        def scale_fn(off, sz):
            rs = RS if sz == 624 else 80

            @pl.when(c == 0)
            def _():
                scale_to(y0_hbm, off, sz, rs)

            @pl.when(c == 1)
            def _():
                scale_to(y1_hbm, off, sz, rs)

        _each_tile_rows(s, zero_fn)
        plsc.subcore_barrier()
        seg_pass(idx_a_hbm, t0_hbm, t1_hbm)
        plsc.subcore_barrier()

        _each_tile_rows(s, scale_fn)
        plsc.subcore_barrier()
        _each_tile_rows(s, zero_fn)
        plsc.subcore_barrier()

        seg_pass(idx_b_hbm, y0_hbm, y1_hbm)
        plsc.subcore_barrier()

        def out_fn(off, sz):
            dst = pl.multiple_of(c * N + off, 8)
            pltpu.sync_copy(acc.at[pl.ds(off, sz)], o_hbm.at[pl.ds(dst, sz)])

        _each_tile_rows(s, out_fn)

    return smooth


_smooth64 = _make_smooth(64, 2)
_smooth32 = _make_smooth(32, 4)


_BR = 1000


def _lin0_body(dvp, dep, x, w, b, o, dvis_o, dei_o):
    dv = dvp[0, :, 0:1] + dvp[1, :, 0:1]
    de = dep[0, :, 0:1] + dep[1, :, 0:1]
    dvis = jnp.where(dv > 0, lax.rsqrt(dv), 0.0)
    dvis_o[...] = dvis
    dei_o[...] = jnp.where(de > 0, 1.0 / de, 0.0)
    h = jnp.dot(x[...], w[...], preferred_element_type=f32) + b[...]
    hs = h * dvis
    o[0] = hs[:, :64]
    o[1] = hs[:, 64:]


_lin0 = pl.pallas_call(
    _lin0_body,
    grid=(N // _BR,),
    in_specs=[
        pl.BlockSpec((2, _BR, DEGW), lambda i: (0, i, 0)),
        pl.BlockSpec((2, _BR, DEGW), lambda i: (0, i, 0)),
        pl.BlockSpec((_BR, DIN), lambda i: (i, 0)),
        pl.BlockSpec((DIN, DH), lambda i: (0, 0)),
        pl.BlockSpec((1, DH), lambda i: (0, 0)),
    ],
    out_specs=[
        pl.BlockSpec((2, _BR, 64), lambda i: (0, i, 0)),
        pl.BlockSpec((_BR, 1), lambda i: (i, 0)),
        pl.BlockSpec((_BR, 1), lambda i: (i, 0)),
    ],
    out_shape=[
        jax.ShapeDtypeStruct((2, N, 64), f32),
        jax.ShapeDtypeStruct((N, 1), f32),
        jax.ShapeDtypeStruct((N, 1), f32),
    ],
)


def _lin1_body(ho, w, b, dvis, o):
    sm = jnp.concatenate([ho[0], ho[1]], axis=1) * dvis[...]
    emb = jnp.maximum(sm, 0.0)
    h2 = jnp.dot(emb, w[...], preferred_element_type=f32) + b[...]
    hs2 = h2 * dvis[...]
    o[0] = hs2[:, :32]
    o[1] = hs2[:, 32:]


_lin1 = pl.pallas_call(
    _lin1_body,
    grid=(N // _BR,),
    in_specs=[
        pl.BlockSpec((2, _BR, 64), lambda i: (0, i, 0)),
        pl.BlockSpec((DH, NCLS), lambda i: (0, 0)),
        pl.BlockSpec((1, NCLS), lambda i: (0, 0)),
        pl.BlockSpec((_BR, 1), lambda i: (i, 0)),
    ],
    out_specs=pl.BlockSpec((2, _BR, 32), lambda i: (0, i, 0)),
    out_shape=jax.ShapeDtypeStruct((2, N, 32), f32),
)


def _final_body(ho, dvis, o):
    o[...] = jnp.concatenate([ho[0], ho[1]], axis=1) * dvis[...]


_final = pl.pallas_call(
    _final_body,
    grid=(N // _BR,),
    in_specs=[
        pl.BlockSpec((2, _BR, 32), lambda i: (0, i, 0)),
        pl.BlockSpec((_BR, 1), lambda i: (i, 0)),
    ],
    out_specs=pl.BlockSpec((_BR, NCLS), lambda i: (i, 0)),
    out_shape=jax.ShapeDtypeStruct((N, NCLS), f32),
)


def kernel(X, incidence, W0, b0, W1, b1):
    v_idx = incidence[0]
    e_idx = incidence[1]
    zNd = jnp.zeros((N, DEGW), f32)
    ones_ch = jnp.ones((CH, DEGW), f32)
    z64 = jnp.zeros((N, 64), f32)
    z32 = jnp.zeros((N, 32), f32)

    v2 = v_idx.reshape(P // CH, CH)
    e2 = e_idx.reshape(P // CH, CH)
    ive = jnp.stack([v2, e2], axis=1)
    iev = jnp.stack([e2, v2], axis=1)

    dvp, dep = _deg(ive, zNd, ones_ch)
    hs, dvis, dei = _lin0(dvp.reshape(2, N, DEGW), dep.reshape(2, N, DEGW),
                          X, W0, b0.reshape(1, DH))
    dei_flat = dei.reshape(N)
    ho, _, _ = _smooth64(hs[0], hs[1], ive, iev, dei_flat, z64)

    hs2 = _lin1(ho.reshape(2, N, 64), W1, b1.reshape(1, NCLS), dvis)
    ho2, _, _ = _smooth32(hs2[0], hs2[1], ive, iev, dei_flat, z32)

    return _final(ho2.reshape(2, N, 32), dvis)
